# baseline (speedup 1.0000x reference)
import jax
import jax.numpy as jnp
from jax.experimental import pallas as pl
from jax.experimental.pallas import tpu as pltpu

N = 4096
IN = 256
OUT = 256
HID = 64
HEADS = 2
BI = 512
HW = 128
H2W = 384


def _dot(a, b):
    return jnp.dot(a, b, preferred_element_type=jnp.float32)


def _leaky(x, slope=0.2):
    return jnp.maximum(x, slope * x)


def _layer_norm(x, g, b, eps=1e-5):
    mu = jnp.mean(x, axis=-1, keepdims=True)
    var = jnp.mean((x - mu) ** 2, axis=-1, keepdims=True)
    return (x - mu) * jax.lax.rsqrt(var + eps) * g + b


def _gelu_exact(x):
    return 0.5 * x * (1.0 + jax.lax.erf(x * (2.0 ** -0.5)))


def _self_loop_weight(g_ref, j):
    gsub = g_ref[:, pl.ds(j * BI, BI)].astype(jnp.float32)
    r = jax.lax.broadcasted_iota(jnp.int32, (BI, BI), 0)
    c = jax.lax.broadcasted_iota(jnp.int32, (BI, BI), 1)
    gdiag = jnp.sum(jnp.where(r == c, gsub, 0.0), axis=1, keepdims=True)
    return 1.0 - gdiag


def _attend(gf, w_self, ci, cj, cjb):
    si = jnp.max(ci)
    shift = _leaky(si + jnp.max(cj))
    sj = shift - si
    eci = jnp.exp(ci - si)
    eci2 = jnp.exp(0.2 * ci - si)
    ecj = jnp.exp(cj - sj).reshape(1, N)
    ecj2 = jnp.exp(0.2 * cj - sj).reshape(1, N)
    p = jnp.maximum(eci * ecj, eci2 * ecj2) * gf
    pd = w_self * jnp.exp(_leaky(ci + cjb) - shift)
    return p, pd


def _fused_kernel(g_ref, x_ref, w0_ref, ai0_ref, aj0_ref, b0_ref,
                  w1_ref, ai1_ref, aj1_ref, b1_ref,
                  ln1g_ref, ln1b_ref, wl_ref, bl_ref,
                  we1_ref, be1_ref, we2_ref, be2_ref,
                  ln2g_ref, ln2b_ref, wo_ref, bo_ref,
                  lnog_ref, lnob_ref, out_ref, h1_s, h2_s):
    ph = pl.program_id(0)
    j = pl.program_id(1)

    @pl.when(ph == 0)
    def _phase0():
        h = _dot(x_ref[:, :], w0_ref[:, :])
        ones = jnp.ones((BI, 1), jnp.float32)
        zeros = jnp.zeros((BI, HW - HID - 1), jnp.float32)
        h1_s[pl.ds(j * BI, BI), :] = jnp.concatenate(
            [h[:, :HID], ones, zeros, h[:, HID:], ones, zeros], axis=-1)

    @pl.when(ph == 1)
    def _phase1():
        gf = g_ref[:, :].astype(jnp.float32)
        w_self = _self_loop_weight(g_ref, j)
        outs = []
        for hd in range(HEADS):
            hh_blk = h1_s[pl.ds(j * BI, BI), hd * HW:hd * HW + HID]
            hblk_aug = h1_s[pl.ds(j * BI, BI), hd * HW:(hd + 1) * HW]
            ci = _dot(hh_blk, ai0_ref[hd, :].reshape(HID, 1))
            cj = _dot(h1_s[:, hd * HW:hd * HW + HID],
                      aj0_ref[hd, :].reshape(HID, 1))
            cjb = _dot(hh_blk, aj0_ref[hd, :].reshape(HID, 1))
            p, pd = _attend(gf, w_self, ci, cj, cjb)
            o_aug = _dot(p, h1_s[:, hd * HW:(hd + 1) * HW]) + pd * hblk_aug
            outs.append(o_aug[:, :HID] / o_aug[:, HID:HID + 1])
        o = jnp.concatenate(outs, axis=-1) + b0_ref[0, :]
        m1 = jnp.where(o > 0, o, jnp.exp(jnp.minimum(o, 0.0)) - 1.0)
        h2 = _dot(m1, w1_ref[:, :])
        ones = jnp.ones((BI, 1), jnp.float32)
        zeros = jnp.zeros((BI, H2W - OUT - 1), jnp.float32)
        h2_s[pl.ds(j * BI, BI), :] = jnp.concatenate(
            [h2, ones, zeros], axis=-1)

    @pl.when(ph == 2)
    def _phase2():
        gf = g_ref[:, :].astype(jnp.float32)
        w_self = _self_loop_weight(g_ref, j)
        h2_blk = h2_s[pl.ds(j * BI, BI), 0:OUT]
        h2aug_blk = h2_s[pl.ds(j * BI, BI), :]
        ci = _dot(h2_blk, ai1_ref[0, :].reshape(OUT, 1))
        cj = _dot(h2_s[:, 0:OUT], aj1_ref[0, :].reshape(OUT, 1))
        cjb = _dot(h2_blk, aj1_ref[0, :].reshape(OUT, 1))
        p, pd = _attend(gf, w_self, ci, cj, cjb)
        o_aug = _dot(p, h2_s[:, :]) + pd * h2aug_blk
        o = o_aug[:, :OUT] / o_aug[:, OUT:OUT + 1] + b1_ref[0, :]

        cat = jnp.concatenate([x_ref[:, :], o], axis=-1)
        x = _layer_norm(cat, ln1g_ref[0, :], ln1b_ref[0, :])
        mm = _dot(x, wl_ref[:, :]) + bl_ref[0, :]
        enc = _dot(_gelu_exact(_dot(mm, we1_ref[:, :]) + be1_ref[0, :]),
                   we2_ref[:, :]) + be2_ref[0, :]
        out = _layer_norm(mm + enc, ln2g_ref[0, :], ln2b_ref[0, :])
        out = _layer_norm(_gelu_exact(_dot(out, wo_ref[:, :]) + bo_ref[0, :]),
                          lnog_ref[0, :], lnob_ref[0, :])
        out_ref[:, :] = out


def _full(shape):
    nd = len(shape)
    return pl.BlockSpec(shape, lambda p, j: (0,) * nd)


def kernel(input, graph, W0, ai0, aj0, b0, W1, ai1, aj1, b1, ln1_g, ln1_b,
           Wl, bl, We1, be1, We2, be2, ln2_g, ln2_b, Wo, bo, lno_g, lno_b):
    nblk = N // BI
    row2 = lambda v: v.reshape(1, -1)

    out = pl.pallas_call(
        _fused_kernel,
        grid=(3, nblk),
        in_specs=[
            pl.BlockSpec((BI, N), lambda p, j: (jnp.where(p == 0, 0, j), 0)),
            pl.BlockSpec((BI, IN), lambda p, j: (j, 0)),
            _full((IN, HEADS * HID)),
            _full((HEADS, HID)),
            _full((HEADS, HID)),
            _full((1, HEADS * HID)),
            _full((HEADS * HID, OUT)),
            _full((1, OUT)),
            _full((1, OUT)),
            _full((1, OUT)),
            _full((1, IN + OUT)),
            _full((1, IN + OUT)),
            _full((IN + OUT, OUT)),
            _full((1, OUT)),
            _full((OUT, OUT)),
            _full((1, OUT)),
            _full((OUT, OUT)),
            _full((1, OUT)),
            _full((1, OUT)),
            _full((1, OUT)),
            _full((OUT, OUT)),
            _full((1, OUT)),
            _full((1, OUT)),
            _full((1, OUT)),
        ],
        out_specs=pl.BlockSpec((BI, OUT), lambda p, j: (j, 0)),
        out_shape=jax.ShapeDtypeStruct((N, OUT), jnp.float32),
        scratch_shapes=[
            pltpu.VMEM((N, HEADS * HW), jnp.float32),
            pltpu.VMEM((N, H2W), jnp.float32),
        ],
    )(graph, input, W0, ai0, aj0, row2(b0), W1, ai1, aj1, row2(b1),
      row2(ln1_g), row2(ln1_b), Wl, row2(bl), We1, row2(be1), We2, row2(be2),
      row2(ln2_g), row2(ln2_b), Wo, row2(bo), row2(lno_g), row2(lno_b))
    return out
